# Initial kernel scaffold; baseline (speedup 1.0000x reference)
import functools

import jax
import jax.numpy as jnp
from jax import lax
from jax.experimental import pallas as pl
from jax.experimental.pallas import tpu as pltpu
from jax.experimental.pallas import tpu_sc as plsc

N = 10000
E = 320000
D = 128
H = 8

NC = 2
NS = 16
NW = NC * NS
EPW = E // NW
RPT = N // NS

_MESH = plsc.VectorSubcoreMesh(
    core_axis_name="c", subcore_axis_name="s", num_cores=NC, num_subcores=NS)

_EPS = 1e-16
_F32 = jnp.float32


def _proj_body(x_ref, wl_ref, wr_ref, xl_ref, xr_ref):
    xv = x_ref[...]
    xl_ref[...] = jnp.dot(xv, wl_ref[...], preferred_element_type=_F32)
    xr_ref[...] = jnp.dot(xv, wr_ref[...], preferred_element_type=_F32)


def _project(x, Wl, Wr, blk=1000):
    n, din = x.shape
    dout = Wl.shape[1]
    return pl.pallas_call(
        _proj_body,
        grid=(n // blk,),
        in_specs=[
            pl.BlockSpec((blk, din), lambda i: (i, 0)),
            pl.BlockSpec((din, dout), lambda i: (0, 0)),
            pl.BlockSpec((din, dout), lambda i: (0, 0)),
        ],
        out_specs=[
            pl.BlockSpec((blk, dout), lambda i: (i, 0)),
            pl.BlockSpec((blk, dout), lambda i: (i, 0)),
        ],
        out_shape=[
            jax.ShapeDtypeStruct((n, dout), _F32),
            jax.ShapeDtypeStruct((n, dout), _F32),
        ],
    )(x, Wl, Wr)


def _act(numer2, denom2, r, b):
    nsum = numer2[0] + numer2[1]
    dsum = jnp.dot(denom2[0] + denom2[1], r,
                   preferred_element_type=_F32) + _EPS
    hv = nsum / dsum + b
    return jnp.where(hv > 0, hv, jnp.expm1(hv))


def _comb_proj_body(n_ref, d_ref, r_ref, b_ref, wl_ref, wr_ref,
                    xl_ref, xr_ref, h_ref):
    hv = _act(n_ref[...], d_ref[...], r_ref[...], b_ref[...])
    h_ref[...] = hv
    xl_ref[...] = jnp.dot(hv, wl_ref[...], preferred_element_type=_F32)
    xr_ref[...] = jnp.dot(hv, wr_ref[...], preferred_element_type=_F32)


def _combine_project(numer, denom, r, b, Wl, Wr, blk=1000):
    n = numer.shape[1]
    din, dout = Wl.shape
    return pl.pallas_call(
        _comb_proj_body,
        grid=(n // blk,),
        in_specs=[
            pl.BlockSpec((2, blk, 128), lambda i: (0, i, 0)),
            pl.BlockSpec((2, blk, 16), lambda i: (0, i, 0)),
            pl.BlockSpec((16, 128), lambda i: (0, 0)),
            pl.BlockSpec((1, 128), lambda i: (0, 0)),
            pl.BlockSpec((din, dout), lambda i: (0, 0)),
            pl.BlockSpec((din, dout), lambda i: (0, 0)),
        ],
        out_specs=[
            pl.BlockSpec((blk, dout), lambda i: (i, 0)),
            pl.BlockSpec((blk, dout), lambda i: (i, 0)),
            pl.BlockSpec((blk, 128), lambda i: (i, 0)),
        ],
        out_shape=[
            jax.ShapeDtypeStruct((n, dout), _F32),
            jax.ShapeDtypeStruct((n, dout), _F32),
            jax.ShapeDtypeStruct((n, 128), _F32),
        ],
    )(numer, denom, r, b, Wl, Wr)


def _comb_res_proj_body(n_ref, d_ref, r_ref, b_ref, res_ref, wl_ref, wr_ref,
                        xl_ref, xr_ref):
    hv = _act(n_ref[...], d_ref[...], r_ref[...], b_ref[...]) + res_ref[...]
    xl_ref[...] = jnp.dot(hv, wl_ref[...], preferred_element_type=_F32)
    xr_ref[...] = jnp.dot(hv, wr_ref[...], preferred_element_type=_F32)


def _combine_res_project(numer, denom, r, b, res, Wl, Wr, blk=1000):
    n = numer.shape[1]
    din, dout = Wl.shape
    return pl.pallas_call(
        _comb_res_proj_body,
        grid=(n // blk,),
        in_specs=[
            pl.BlockSpec((2, blk, 128), lambda i: (0, i, 0)),
            pl.BlockSpec((2, blk, 16), lambda i: (0, i, 0)),
            pl.BlockSpec((16, 128), lambda i: (0, 0)),
            pl.BlockSpec((1, 128), lambda i: (0, 0)),
            pl.BlockSpec((blk, 128), lambda i: (i, 0)),
            pl.BlockSpec((din, dout), lambda i: (0, 0)),
            pl.BlockSpec((din, dout), lambda i: (0, 0)),
        ],
        out_specs=[
            pl.BlockSpec((blk, dout), lambda i: (i, 0)),
            pl.BlockSpec((blk, dout), lambda i: (i, 0)),
        ],
        out_shape=[
            jax.ShapeDtypeStruct((n, dout), _F32),
            jax.ShapeDtypeStruct((n, dout), _F32),
        ],
    )(numer, denom, r, b, res, Wl, Wr)


def _final_body(p_ref, b_ref, o_ref):
    o_ref[...] = (p_ref[0] + p_ref[1]) * (1.0 / H) + b_ref[...]


def _final(partials, b, blk=1000):
    n = partials.shape[1]
    return pl.pallas_call(
        _final_body,
        grid=(n // blk,),
        in_specs=[
            pl.BlockSpec((2, blk, 128), lambda i: (0, i, 0)),
            pl.BlockSpec((1, 128), lambda i: (0, 0)),
        ],
        out_specs=pl.BlockSpec((blk, 128), lambda i: (i, 0)),
        out_shape=jax.ShapeDtypeStruct((n, 128), _F32),
    )(partials, b)


def _wid():
    return lax.axis_index("s") * NC + lax.axis_index("c")


K1 = 80
K2 = 40


@functools.partial(
    pl.kernel,
    out_type=[
        jax.ShapeDtypeStruct((NC, N, 128), _F32),
        jax.ShapeDtypeStruct((NC, N, 16), _F32),
    ],
    mesh=_MESH,
    scratch_types=[
        pltpu.VMEM((K1,), jnp.int32),
        pltpu.VMEM((K1,), jnp.int32),
        pltpu.VMEM((K1, 128), _F32),
        pltpu.VMEM((K1, 128), _F32),
        pltpu.VMEM((K1, 128), _F32),
        pltpu.VMEM((K1, 16), _F32),
        pltpu.VMEM((H, 16), _F32),
        pltpu.VMEM_SHARED((N, 128), _F32),
        pltpu.VMEM_SHARED((N, 16), _F32),
        pltpu.SemaphoreType.DMA,
        pltpu.SemaphoreType.DMA,
    ],
)
def _edge16(xl_hbm, xr_hbm, src_hbm, dst_hbm, att_hbm, z128_hbm, z16_hbm,
            numer_out, denom_out,
            sidx, didx, xlb, xrb, msgb, dvb, attv, snum, sden, sem1, sem2):
    cid = lax.axis_index("c")
    sid = lax.axis_index("s")
    wid = _wid()

    r0 = sid * RPT
    pltpu.sync_copy(z128_hbm, snum.at[pl.ds(r0, RPT)])
    pltpu.sync_copy(z16_hbm, sden.at[pl.ds(r0, RPT)])
    pltpu.sync_copy(att_hbm, attv)
    plsc.subcore_barrier()

    lane = lax.iota(jnp.int32, 16)

    def chunk(j, carry):
        base = wid * EPW + j * K1
        pltpu.sync_copy(src_hbm.at[pl.ds(base, K1)], sidx)
        pltpu.sync_copy(dst_hbm.at[pl.ds(base, K1)], didx)
        cp1 = pltpu.async_copy(xl_hbm.at[sidx], xlb, sem1)
        cp2 = pltpu.async_copy(xr_hbm.at[didx], xrb, sem2)
        cp1.wait()
        cp2.wait()

        def edge(e, c2):
            dv = jnp.zeros((16,), _F32)
            for h in range(H):
                a = xlb[e, pl.ds(h * 16, 16)]
                r = xrb[e, pl.ds(h * 16, 16)]
                t = a + r
                t = jnp.where(t > 0, t, 0.2 * t)
                s = jnp.sum(attv[h, :] * t)
                ev = jnp.exp(jnp.broadcast_to(s, (16,)))
                msgb[e, pl.ds(h * 16, 16)] = ev * a
                dv = dv + jnp.where(lane == h, ev, 0.0)
            dvb[e, :] = dv
            return c2

        lax.fori_loop(0, K1, edge, 0)
        pltpu.sync_copy(msgb, snum.at[didx], add=True)
        pltpu.sync_copy(dvb, sden.at[didx], add=True)
        return carry

    lax.fori_loop(0, EPW // K1, chunk, 0)
    plsc.subcore_barrier()

    pltpu.sync_copy(snum.at[pl.ds(r0, RPT)], numer_out.at[cid, pl.ds(r0, RPT)])
    pltpu.sync_copy(sden.at[pl.ds(r0, RPT)], denom_out.at[cid, pl.ds(r0, RPT)])


@functools.partial(
    pl.kernel,
    out_type=[
        jax.ShapeDtypeStruct((E, 16), _F32),
        jax.ShapeDtypeStruct((NC, N, 16), _F32),
    ],
    mesh=_MESH,
    scratch_types=[
        pltpu.VMEM((K2,), jnp.int32),
        pltpu.VMEM((K2,), jnp.int32),
        pltpu.VMEM((K2, 1024), _F32),
        pltpu.VMEM((K2, 1024), _F32),
        pltpu.VMEM((K2, 16), _F32),
        pltpu.VMEM((H, 128), _F32),
        pltpu.VMEM_SHARED((N, 16), _F32),
        pltpu.SemaphoreType.DMA,
        pltpu.SemaphoreType.DMA,
    ],
)
def _edge2a(xl_hbm, xr_hbm, src_hbm, dst_hbm, att_hbm, z16_hbm,
            ex_out, denom_out,
            sidx, didx, xlb, xrb, dvb, attv, sden, sem1, sem2):
    cid = lax.axis_index("c")
    sid = lax.axis_index("s")
    wid = _wid()

    r0 = sid * RPT
    pltpu.sync_copy(z16_hbm, sden.at[pl.ds(r0, RPT)])
    pltpu.sync_copy(att_hbm, attv)
    plsc.subcore_barrier()

    lane = lax.iota(jnp.int32, 16)

    def chunk(j, carry):
        base = wid * EPW + j * K2
        pltpu.sync_copy(src_hbm.at[pl.ds(base, K2)], sidx)
        pltpu.sync_copy(dst_hbm.at[pl.ds(base, K2)], didx)
        cp1 = pltpu.async_copy(xl_hbm.at[sidx], xlb, sem1)
        cp2 = pltpu.async_copy(xr_hbm.at[didx], xrb, sem2)
        cp1.wait()
        cp2.wait()

        def edge(e, c2):
            dv = jnp.zeros((16,), _F32)
            for h in range(H):
                acc = jnp.zeros((16,), _F32)
                for cb in range(8):
                    o = h * 128 + cb * 16
                    a = xlb[e, pl.ds(o, 16)]
                    r = xrb[e, pl.ds(o, 16)]
                    t = a + r
                    t = jnp.where(t > 0, t, 0.2 * t)
                    acc = acc + attv[h, pl.ds(cb * 16, 16)] * t
                s = jnp.sum(acc)
                ev = jnp.exp(jnp.broadcast_to(s, (16,)))
                dv = dv + jnp.where(lane == h, ev, 0.0)
            dvb[e, :] = dv
            return c2

        lax.fori_loop(0, K2, edge, 0)
        pltpu.sync_copy(dvb, ex_out.at[pl.ds(base, K2)])
        pltpu.sync_copy(dvb, sden.at[didx], add=True)
        return carry

    lax.fori_loop(0, EPW // K2, chunk, 0)
    plsc.subcore_barrier()

    pltpu.sync_copy(sden.at[pl.ds(r0, RPT)], denom_out.at[cid, pl.ds(r0, RPT)])


@functools.partial(
    pl.kernel,
    out_type=jax.ShapeDtypeStruct((NC, N, 128), _F32),
    mesh=_MESH,
    scratch_types=[
        pltpu.VMEM((K2,), jnp.int32),
        pltpu.VMEM((K2,), jnp.int32),
        pltpu.VMEM((K2, 1024), _F32),
        pltpu.VMEM((K2, 16), _F32),
        pltpu.VMEM((K2, 16), _F32),
        pltpu.VMEM((K2, 16), _F32),
        pltpu.VMEM((K2, 128), _F32),
        pltpu.VMEM_SHARED((N, 128), _F32),
        pltpu.SemaphoreType.DMA,
        pltpu.SemaphoreType.DMA,
        pltpu.SemaphoreType.DMA,
    ],
)
def _edge2b(xl_hbm, src_hbm, dst_hbm, ex_hbm, d0_hbm, d1_hbm, z128_hbm,
            out_hbm,
            sidx, didx, xlb, exb, d0b, d1b, msgb, sout, sem1, sem2, sem3):
    cid = lax.axis_index("c")
    sid = lax.axis_index("s")
    wid = _wid()

    r0 = sid * RPT
    pltpu.sync_copy(z128_hbm, sout.at[pl.ds(r0, RPT)])
    plsc.subcore_barrier()

    def chunk(j, carry):
        base = wid * EPW + j * K2
        pltpu.sync_copy(src_hbm.at[pl.ds(base, K2)], sidx)
        pltpu.sync_copy(dst_hbm.at[pl.ds(base, K2)], didx)
        cp1 = pltpu.async_copy(xl_hbm.at[sidx], xlb, sem1)
        cp2 = pltpu.async_copy(d0_hbm.at[didx], d0b, sem2)
        cp3 = pltpu.async_copy(d1_hbm.at[didx], d1b, sem3)
        pltpu.sync_copy(ex_hbm.at[pl.ds(base, K2)], exb)
        cp1.wait()
        cp2.wait()
        cp3.wait()

        def edge(e, c2):
            ex = exb[e, :]
            dn = d0b[e, :] + d1b[e, :] + _EPS
            w = ex / dn
            m = [jnp.zeros((16,), _F32) for _ in range(8)]
            for h in range(H):
                wh = w[h]
                for cb in range(8):
                    m[cb] = m[cb] + wh * xlb[e, pl.ds(h * 128 + cb * 16, 16)]
            for cb in range(8):
                msgb[e, pl.ds(cb * 16, 16)] = m[cb]
            return c2

        lax.fori_loop(0, K2, edge, 0)
        pltpu.sync_copy(msgb, sout.at[didx], add=True)
        return carry

    lax.fori_loop(0, EPW // K2, chunk, 0)
    plsc.subcore_barrier()

    pltpu.sync_copy(sout.at[pl.ds(r0, RPT)], out_hbm.at[cid, pl.ds(r0, RPT)])


def kernel(x, edge_index, Wl0, Wr0, att0, b0, Wl1, Wr1, att1, b1,
           Wl2, Wr2, att2, b2):
    src = edge_index[0]
    dst = edge_index[1]

    r = (jnp.arange(128)[None, :] // 16 ==
         jnp.arange(16)[:, None]).astype(_F32)
    z128 = jnp.zeros((RPT, 128), _F32)
    z16 = jnp.zeros((RPT, 16), _F32)
    b0r = b0.reshape(1, 128)
    b1r = b1.reshape(1, 128)
    b2r = b2.reshape(1, 128)

    xl0, xr0 = _project(x, Wl0, Wr0)
    n0, d0 = _edge16(xl0, xr0, src, dst, att0, z128, z16)
    xl1, xr1, hin1 = _combine_project(n0, d0, r, b0r, Wl1, Wr1)
    n1, d1 = _edge16(xl1, xr1, src, dst, att1, z128, z16)
    xl2, xr2 = _combine_res_project(n1, d1, r, b1r, hin1, Wl2, Wr2)
    exb, dp = _edge2a(xl2, xr2, src, dst, att2, z16)
    outp = _edge2b(xl2, src, dst, exb, dp[0], dp[1], z128)
    return _final(outp, b2r)


# speedup vs baseline: 12.7046x; 12.7046x over previous
import functools

import jax
import jax.numpy as jnp
from jax import lax
from jax.experimental import pallas as pl
from jax.experimental.pallas import tpu as pltpu
from jax.experimental.pallas import tpu_sc as plsc

N = 10000
E = 320000
D = 128
H = 8

NC = 2
NS = 16
NW = NC * NS
EPW = E // NW
RA = 624
TAIL = N - NS * RA

_MESH = plsc.VectorSubcoreMesh(
    core_axis_name="c", subcore_axis_name="s", num_cores=NC, num_subcores=NS)

_EPS = 1e-16
_F32 = jnp.float32


def _proj_body(x_ref, wl_ref, wr_ref, xl_ref, xr_ref):
    xv = x_ref[...]
    xl_ref[...] = jnp.dot(xv, wl_ref[...], preferred_element_type=_F32)
    xr_ref[...] = jnp.dot(xv, wr_ref[...], preferred_element_type=_F32)


def _project(x, Wl, Wr, blk=1000):
    n, din = x.shape
    dout = Wl.shape[1]
    return pl.pallas_call(
        _proj_body,
        grid=(n // blk,),
        in_specs=[
            pl.BlockSpec((blk, din), lambda i: (i, 0)),
            pl.BlockSpec((din, dout), lambda i: (0, 0)),
            pl.BlockSpec((din, dout), lambda i: (0, 0)),
        ],
        out_specs=[
            pl.BlockSpec((blk, dout), lambda i: (i, 0)),
            pl.BlockSpec((blk, dout), lambda i: (i, 0)),
        ],
        out_shape=[
            jax.ShapeDtypeStruct((n, dout), _F32),
            jax.ShapeDtypeStruct((n, dout), _F32),
        ],
    )(x, Wl, Wr)


def _act(numer2, denom2, r, b):
    nsum = numer2[0] + numer2[1]
    dsum = jnp.dot((denom2[0] + denom2[1])[:, :16], r,
                   preferred_element_type=_F32) + _EPS
    hv = nsum / dsum + b
    return jnp.where(hv > 0, hv, jnp.exp(hv) - 1.0)


def _comb_proj_body(n_ref, d_ref, r_ref, b_ref, wl_ref, wr_ref,
                    xl_ref, xr_ref, h_ref):
    hv = _act(n_ref[...], d_ref[...], r_ref[...], b_ref[...])
    h_ref[...] = hv
    xl_ref[...] = jnp.dot(hv, wl_ref[...], preferred_element_type=_F32)
    xr_ref[...] = jnp.dot(hv, wr_ref[...], preferred_element_type=_F32)


def _combine_project(numer, denom, r, b, Wl, Wr, blk=1000):
    n = numer.shape[1]
    din, dout = Wl.shape
    return pl.pallas_call(
        _comb_proj_body,
        grid=(n // blk,),
        in_specs=[
            pl.BlockSpec((2, blk, 128), lambda i: (0, i, 0)),
            pl.BlockSpec((2, blk, 128), lambda i: (0, i, 0)),
            pl.BlockSpec((16, 128), lambda i: (0, 0)),
            pl.BlockSpec((1, 128), lambda i: (0, 0)),
            pl.BlockSpec((din, dout), lambda i: (0, 0)),
            pl.BlockSpec((din, dout), lambda i: (0, 0)),
        ],
        out_specs=[
            pl.BlockSpec((blk, dout), lambda i: (i, 0)),
            pl.BlockSpec((blk, dout), lambda i: (i, 0)),
            pl.BlockSpec((blk, 128), lambda i: (i, 0)),
        ],
        out_shape=[
            jax.ShapeDtypeStruct((n, dout), _F32),
            jax.ShapeDtypeStruct((n, dout), _F32),
            jax.ShapeDtypeStruct((n, 128), _F32),
        ],
    )(numer, denom, r, b, Wl, Wr)


def _comb_res_proj_body(n_ref, d_ref, r_ref, b_ref, res_ref, wl_ref, wr_ref,
                        xl_ref, xr_ref):
    hv = _act(n_ref[...], d_ref[...], r_ref[...], b_ref[...]) + res_ref[...]
    xl_ref[...] = jnp.dot(hv, wl_ref[...], preferred_element_type=_F32)
    xr_ref[...] = jnp.dot(hv, wr_ref[...], preferred_element_type=_F32)


def _combine_res_project(numer, denom, r, b, res, Wl, Wr, blk=1000):
    n = numer.shape[1]
    din, dout = Wl.shape
    return pl.pallas_call(
        _comb_res_proj_body,
        grid=(n // blk,),
        in_specs=[
            pl.BlockSpec((2, blk, 128), lambda i: (0, i, 0)),
            pl.BlockSpec((2, blk, 128), lambda i: (0, i, 0)),
            pl.BlockSpec((16, 128), lambda i: (0, 0)),
            pl.BlockSpec((1, 128), lambda i: (0, 0)),
            pl.BlockSpec((blk, 128), lambda i: (i, 0)),
            pl.BlockSpec((din, dout), lambda i: (0, 0)),
            pl.BlockSpec((din, dout), lambda i: (0, 0)),
        ],
        out_specs=[
            pl.BlockSpec((blk, dout), lambda i: (i, 0)),
            pl.BlockSpec((blk, dout), lambda i: (i, 0)),
        ],
        out_shape=[
            jax.ShapeDtypeStruct((n, dout), _F32),
            jax.ShapeDtypeStruct((n, dout), _F32),
        ],
    )(numer, denom, r, b, res, Wl, Wr)


def _dsum_body(d_ref, o_ref):
    o_ref[...] = d_ref[0] + d_ref[1]


def _dsum(denom, blk=1000):
    n = denom.shape[1]
    return pl.pallas_call(
        _dsum_body,
        grid=(n // blk,),
        in_specs=[pl.BlockSpec((2, blk, 128), lambda i: (0, i, 0))],
        out_specs=pl.BlockSpec((blk, 128), lambda i: (i, 0)),
        out_shape=jax.ShapeDtypeStruct((n, 128), _F32),
    )(denom)


def _final_body(p_ref, b_ref, o_ref):
    o_ref[...] = (p_ref[0] + p_ref[1]) * (1.0 / H) + b_ref[...]


def _final(partials, b, blk=1000):
    n = partials.shape[1]
    return pl.pallas_call(
        _final_body,
        grid=(n // blk,),
        in_specs=[
            pl.BlockSpec((2, blk, 128), lambda i: (0, i, 0)),
            pl.BlockSpec((1, 128), lambda i: (0, 0)),
        ],
        out_specs=pl.BlockSpec((blk, 128), lambda i: (i, 0)),
        out_shape=jax.ShapeDtypeStruct((n, 128), _F32),
    )(partials, b)


def _wid():
    return lax.axis_index("s") * NC + lax.axis_index("c")


def _lane_sum(v):
    for stride in (8, 4, 2, 1):
        idx = lax.iota(jnp.int32, 16) ^ stride
        v = v + v.at[idx].get(mode="promise_in_bounds")
    return v


def _zero_shared(z_hbm, shared, sid):
    pltpu.sync_copy(z_hbm, shared.at[pl.ds(sid * RA, RA)])

    @pl.when(sid == NS - 1)
    def _tail():
        pltpu.sync_copy(z_hbm.at[pl.ds(0, TAIL)],
                        shared.at[pl.ds(NS * RA, TAIL)])


def _dump_shared(shared, out_hbm, cid, sid):
    r0 = sid * RA
    pltpu.sync_copy(shared.at[pl.ds(r0, RA)], out_hbm.at[cid, pl.ds(r0, RA)])

    @pl.when(sid == NS - 1)
    def _tail():
        pltpu.sync_copy(shared.at[pl.ds(NS * RA, TAIL)],
                        out_hbm.at[cid, pl.ds(NS * RA, TAIL)])


K1 = 40
K2A = 40
K2B = 16
KD = 80


@functools.partial(
    pl.kernel,
    out_type=[
        jax.ShapeDtypeStruct((NC, N, 128), _F32),
        jax.ShapeDtypeStruct((E * 16,), _F32),
    ],
    mesh=_MESH,
    scratch_types=[
        pltpu.VMEM((K1,), jnp.int32),
        pltpu.VMEM((K1,), jnp.int32),
        pltpu.VMEM((K1, 128), _F32),
        pltpu.VMEM((K1, 128), _F32),
        pltpu.VMEM((K1, 128), _F32),
        pltpu.VMEM((K1 * 16,), _F32),
        pltpu.VMEM((H, 16), _F32),
        pltpu.VMEM_SHARED((N, 128), _F32),
        pltpu.SemaphoreType.DMA,
        pltpu.SemaphoreType.DMA,
    ],
)
def _edge16(xl_hbm, xr_hbm, src_hbm, dst_hbm, att_hbm, z128_hbm,
            numer_out, ex_out,
            sidx, didx, xlb, xrb, msgb, exb, attv, snum, sem1, sem2):
    cid = lax.axis_index("c")
    sid = lax.axis_index("s")
    wid = _wid()

    _zero_shared(z128_hbm, snum, sid)
    pltpu.sync_copy(att_hbm, attv)
    plsc.subcore_barrier()

    lane = lax.iota(jnp.int32, 16)

    def chunk(j, carry):
        base = wid * EPW + j * K1
        pltpu.sync_copy(src_hbm.at[pl.ds(base, K1)], sidx)
        pltpu.sync_copy(dst_hbm.at[pl.ds(base, K1)], didx)
        cp1 = pltpu.async_copy(xl_hbm.at[sidx], xlb, sem1)
        cp2 = pltpu.async_copy(xr_hbm.at[didx], xrb, sem2)
        cp1.wait()
        cp2.wait()

        def edge(e, c2):
            dv = jnp.zeros((16,), _F32)
            for h in range(H):
                a = xlb[e, pl.ds(h * 16, 16)]
                r = xrb[e, pl.ds(h * 16, 16)]
                t = a + r
                t = jnp.where(t > 0, t, 0.2 * t)
                ev = jnp.exp(_lane_sum(attv[h, :] * t))
                msgb[e, pl.ds(h * 16, 16)] = ev * a
                dv = dv + jnp.where(lane == h, ev, 0.0)
            exb[pl.ds(e * 16, 16)] = dv
            return c2

        lax.fori_loop(0, K1, edge, 0)
        pltpu.sync_copy(msgb, snum.at[didx], add=True)
        pltpu.sync_copy(exb, ex_out.at[pl.ds(base * 16, K1 * 16)])
        return carry

    lax.fori_loop(0, EPW // K1, chunk, 0)
    plsc.subcore_barrier()

    _dump_shared(snum, numer_out, cid, sid)


@functools.partial(
    pl.kernel,
    out_type=jax.ShapeDtypeStruct((NC, N, 128), _F32),
    mesh=_MESH,
    scratch_types=[
        pltpu.VMEM((KD,), jnp.int32),
        pltpu.VMEM((KD * 16,), _F32),
        pltpu.VMEM((KD, 128), _F32),
        pltpu.VMEM_SHARED((N, 128), _F32),
        pltpu.SemaphoreType.DMA,
    ],
)
def _denacc(dst_hbm, ex_hbm, z128_hbm,
            denom_out,
            didx, exb, dwb, sden, sem1):
    cid = lax.axis_index("c")
    sid = lax.axis_index("s")
    wid = _wid()

    _zero_shared(z128_hbm, sden, sid)

    zero16 = jnp.zeros((16,), _F32)

    def zrow(e, carry):
        for cb in range(8):
            dwb[e, pl.ds(cb * 16, 16)] = zero16
        return carry

    lax.fori_loop(0, KD, zrow, 0)
    plsc.subcore_barrier()

    def chunk(j, carry):
        base = wid * EPW + j * KD
        pltpu.sync_copy(dst_hbm.at[pl.ds(base, KD)], didx)
        pltpu.sync_copy(ex_hbm.at[pl.ds(base * 16, KD * 16)], exb)

        def edge(e, c2):
            dwb[e, pl.ds(0, 16)] = exb[pl.ds(e * 16, 16)]
            return c2

        lax.fori_loop(0, KD, edge, 0)
        pltpu.sync_copy(dwb, sden.at[didx], add=True)
        return carry

    lax.fori_loop(0, EPW // KD, chunk, 0)
    plsc.subcore_barrier()

    _dump_shared(sden, denom_out, cid, sid)


@functools.partial(
    pl.kernel,
    out_type=jax.ShapeDtypeStruct((E * 16,), _F32),
    mesh=_MESH,
    scratch_types=[
        pltpu.VMEM((K2A,), jnp.int32),
        pltpu.VMEM((K2A,), jnp.int32),
        pltpu.VMEM((K2A, 1024), _F32),
        pltpu.VMEM((K2A, 1024), _F32),
        pltpu.VMEM((K2A * 16,), _F32),
        pltpu.VMEM((H, 128), _F32),
        pltpu.SemaphoreType.DMA,
        pltpu.SemaphoreType.DMA,
    ],
)
def _edge2a(xl_hbm, xr_hbm, src_hbm, dst_hbm, att_hbm,
            ex_out,
            sidx, didx, xlb, xrb, exb, attv, sem1, sem2):
    wid = _wid()

    pltpu.sync_copy(att_hbm, attv)

    lane = lax.iota(jnp.int32, 16)

    def chunk(j, carry):
        base = wid * EPW + j * K2A
        pltpu.sync_copy(src_hbm.at[pl.ds(base, K2A)], sidx)
        pltpu.sync_copy(dst_hbm.at[pl.ds(base, K2A)], didx)
        cp1 = pltpu.async_copy(xl_hbm.at[sidx], xlb, sem1)
        cp2 = pltpu.async_copy(xr_hbm.at[didx], xrb, sem2)
        cp1.wait()
        cp2.wait()

        def edge(e, c2):
            dv = jnp.zeros((16,), _F32)
            for h in range(H):
                acc = jnp.zeros((16,), _F32)
                for cb in range(8):
                    o = h * 128 + cb * 16
                    a = xlb[e, pl.ds(o, 16)]
                    r = xrb[e, pl.ds(o, 16)]
                    t = a + r
                    t = jnp.where(t > 0, t, 0.2 * t)
                    acc = acc + attv[h, pl.ds(cb * 16, 16)] * t
                ev = jnp.exp(_lane_sum(acc))
                dv = dv + jnp.where(lane == h, ev, 0.0)
            exb[pl.ds(e * 16, 16)] = dv
            return c2

        lax.fori_loop(0, K2A, edge, 0)
        pltpu.sync_copy(exb, ex_out.at[pl.ds(base * 16, K2A * 16)])
        return carry

    lax.fori_loop(0, EPW // K2A, chunk, 0)


@functools.partial(
    pl.kernel,
    out_type=jax.ShapeDtypeStruct((NC, N, 128), _F32),
    mesh=_MESH,
    scratch_types=[
        pltpu.VMEM((K2B,), jnp.int32),
        pltpu.VMEM((K2B,), jnp.int32),
        pltpu.VMEM((K2B, 1024), _F32),
        pltpu.VMEM((K2B * 16,), _F32),
        pltpu.VMEM((K2B, 128), _F32),
        pltpu.VMEM((K2B, 128), _F32),
        pltpu.VMEM_SHARED((N, 128), _F32),
        pltpu.SemaphoreType.DMA,
        pltpu.SemaphoreType.DMA,
    ],
)
def _edge2b(xl_hbm, src_hbm, dst_hbm, ex_hbm, dn_hbm, z128_hbm,
            out_hbm,
            sidx, didx, xlb, exb, dnb, msgb, sout, sem1, sem2):
    cid = lax.axis_index("c")
    sid = lax.axis_index("s")
    wid = _wid()

    _zero_shared(z128_hbm, sout, sid)
    plsc.subcore_barrier()

    def chunk(j, carry):
        base = wid * EPW + j * K2B
        pltpu.sync_copy(src_hbm.at[pl.ds(base, K2B)], sidx)
        pltpu.sync_copy(dst_hbm.at[pl.ds(base, K2B)], didx)
        cp1 = pltpu.async_copy(xl_hbm.at[sidx], xlb, sem1)
        cp2 = pltpu.async_copy(dn_hbm.at[didx], dnb, sem2)
        pltpu.sync_copy(ex_hbm.at[pl.ds(base * 16, K2B * 16)], exb)
        cp1.wait()
        cp2.wait()

        def edge(e, c2):
            ex = exb[pl.ds(e * 16, 16)]
            dn = dnb[e, pl.ds(0, 16)] + _EPS
            w = ex / dn
            m = [jnp.zeros((16,), _F32) for _ in range(8)]
            for h in range(H):
                whv = jnp.broadcast_to(w[h], (16,))
                for cb in range(8):
                    m[cb] = m[cb] + whv * xlb[e, pl.ds(h * 128 + cb * 16, 16)]
            for cb in range(8):
                msgb[e, pl.ds(cb * 16, 16)] = m[cb]
            return c2

        lax.fori_loop(0, K2B, edge, 0)
        pltpu.sync_copy(msgb, sout.at[didx], add=True)
        return carry

    lax.fori_loop(0, EPW // K2B, chunk, 0)
    plsc.subcore_barrier()

    _dump_shared(sout, out_hbm, cid, sid)


def kernel(x, edge_index, Wl0, Wr0, att0, b0, Wl1, Wr1, att1, b1,
           Wl2, Wr2, att2, b2):
    src = edge_index[0]
    dst = edge_index[1]

    r = (jnp.arange(128)[None, :] // 16 ==
         jnp.arange(16)[:, None]).astype(_F32)
    z128 = jnp.zeros((RA, 128), _F32)
    b0r = b0.reshape(1, 128)
    b1r = b1.reshape(1, 128)
    b2r = b2.reshape(1, 128)

    xl0, xr0 = _project(x, Wl0, Wr0)
    n0, ex0 = _edge16(xl0, xr0, src, dst, att0, z128)
    d0 = _denacc(dst, ex0, z128)
    xl1, xr1, hin1 = _combine_project(n0, d0, r, b0r, Wl1, Wr1)
    n1, ex1 = _edge16(xl1, xr1, src, dst, att1, z128)
    d1 = _denacc(dst, ex1, z128)
    xl2, xr2 = _combine_res_project(n1, d1, r, b1r, hin1, Wl2, Wr2)
    ex2 = _edge2a(xl2, xr2, src, dst, att2)
    d2 = _denacc(dst, ex2, z128)
    dn2 = _dsum(d2)
    outp = _edge2b(xl2, src, dst, ex2, dn2, z128)
    return _final(outp, b2r)


# speedup vs baseline: 18.7417x; 1.4752x over previous
import functools

import jax
import jax.numpy as jnp
from jax import lax
from jax.experimental import pallas as pl
from jax.experimental.pallas import tpu as pltpu
from jax.experimental.pallas import tpu_sc as plsc

N = 10000
E = 320000
D = 128
H = 8

NC = 2
NS = 16
NW = NC * NS
EPW = E // NW
RA = 624
TAIL = N - NS * RA

_MESH = plsc.VectorSubcoreMesh(
    core_axis_name="c", subcore_axis_name="s", num_cores=NC, num_subcores=NS)

_EPS = 1e-16
_F32 = jnp.float32


def _proj_body(x_ref, wl_ref, wr_ref, xl_ref, xr_ref):
    xv = x_ref[...]
    xl_ref[...] = jnp.dot(xv, wl_ref[...], preferred_element_type=_F32)
    xr_ref[...] = jnp.dot(xv, wr_ref[...], preferred_element_type=_F32)


def _project(x, Wl, Wr, blk=1000):
    n, din = x.shape
    dout = Wl.shape[1]
    return pl.pallas_call(
        _proj_body,
        grid=(n // blk,),
        in_specs=[
            pl.BlockSpec((blk, din), lambda i: (i, 0)),
            pl.BlockSpec((din, dout), lambda i: (0, 0)),
            pl.BlockSpec((din, dout), lambda i: (0, 0)),
        ],
        out_specs=[
            pl.BlockSpec((blk, dout), lambda i: (i, 0)),
            pl.BlockSpec((blk, dout), lambda i: (i, 0)),
        ],
        out_shape=[
            jax.ShapeDtypeStruct((n, dout), _F32),
            jax.ShapeDtypeStruct((n, dout), _F32),
        ],
    )(x, Wl, Wr)


def _act(numer2, denom2, r, b):
    nsum = numer2[0] + numer2[1]
    dsum = jnp.dot((denom2[0] + denom2[1])[:, :16], r,
                   preferred_element_type=_F32) + _EPS
    hv = nsum / dsum + b
    return jnp.where(hv > 0, hv, jnp.exp(hv) - 1.0)


def _comb_proj_body(n_ref, d_ref, r_ref, b_ref, wl_ref, wr_ref,
                    xl_ref, xr_ref, h_ref):
    hv = _act(n_ref[...], d_ref[...], r_ref[...], b_ref[...])
    h_ref[...] = hv
    xl_ref[...] = jnp.dot(hv, wl_ref[...], preferred_element_type=_F32)
    xr_ref[...] = jnp.dot(hv, wr_ref[...], preferred_element_type=_F32)


def _combine_project(numer, denom, r, b, Wl, Wr, blk=1000):
    n = numer.shape[1]
    din, dout = Wl.shape
    return pl.pallas_call(
        _comb_proj_body,
        grid=(n // blk,),
        in_specs=[
            pl.BlockSpec((2, blk, 128), lambda i: (0, i, 0)),
            pl.BlockSpec((2, blk, 128), lambda i: (0, i, 0)),
            pl.BlockSpec((16, 128), lambda i: (0, 0)),
            pl.BlockSpec((1, 128), lambda i: (0, 0)),
            pl.BlockSpec((din, dout), lambda i: (0, 0)),
            pl.BlockSpec((din, dout), lambda i: (0, 0)),
        ],
        out_specs=[
            pl.BlockSpec((blk, dout), lambda i: (i, 0)),
            pl.BlockSpec((blk, dout), lambda i: (i, 0)),
            pl.BlockSpec((blk, 128), lambda i: (i, 0)),
        ],
        out_shape=[
            jax.ShapeDtypeStruct((n, dout), _F32),
            jax.ShapeDtypeStruct((n, dout), _F32),
            jax.ShapeDtypeStruct((n, 128), _F32),
        ],
    )(numer, denom, r, b, Wl, Wr)


def _comb_res_proj_body(n_ref, d_ref, r_ref, b_ref, res_ref, wl_ref, wr_ref,
                        xl_ref, xr_ref):
    hv = _act(n_ref[...], d_ref[...], r_ref[...], b_ref[...]) + res_ref[...]
    xl_ref[...] = jnp.dot(hv, wl_ref[...], preferred_element_type=_F32)
    xr_ref[...] = jnp.dot(hv, wr_ref[...], preferred_element_type=_F32)


def _combine_res_project(numer, denom, r, b, res, Wl, Wr, blk=1000):
    n = numer.shape[1]
    din, dout = Wl.shape
    return pl.pallas_call(
        _comb_res_proj_body,
        grid=(n // blk,),
        in_specs=[
            pl.BlockSpec((2, blk, 128), lambda i: (0, i, 0)),
            pl.BlockSpec((2, blk, 128), lambda i: (0, i, 0)),
            pl.BlockSpec((16, 128), lambda i: (0, 0)),
            pl.BlockSpec((1, 128), lambda i: (0, 0)),
            pl.BlockSpec((blk, 128), lambda i: (i, 0)),
            pl.BlockSpec((din, dout), lambda i: (0, 0)),
            pl.BlockSpec((din, dout), lambda i: (0, 0)),
        ],
        out_specs=[
            pl.BlockSpec((blk, dout), lambda i: (i, 0)),
            pl.BlockSpec((blk, dout), lambda i: (i, 0)),
        ],
        out_shape=[
            jax.ShapeDtypeStruct((n, dout), _F32),
            jax.ShapeDtypeStruct((n, dout), _F32),
        ],
    )(numer, denom, r, b, res, Wl, Wr)


def _dsum_body(d_ref, o_ref):
    o_ref[...] = d_ref[0] + d_ref[1]


def _dsum(denom, blk=1000):
    n = denom.shape[1]
    return pl.pallas_call(
        _dsum_body,
        grid=(n // blk,),
        in_specs=[pl.BlockSpec((2, blk, 128), lambda i: (0, i, 0))],
        out_specs=pl.BlockSpec((blk, 128), lambda i: (i, 0)),
        out_shape=jax.ShapeDtypeStruct((n, 128), _F32),
    )(denom)


def _final_body(p_ref, b_ref, o_ref):
    o_ref[...] = (p_ref[0] + p_ref[1]) * (1.0 / H) + b_ref[...]


def _final(partials, b, blk=1000):
    n = partials.shape[1]
    return pl.pallas_call(
        _final_body,
        grid=(n // blk,),
        in_specs=[
            pl.BlockSpec((2, blk, 128), lambda i: (0, i, 0)),
            pl.BlockSpec((1, 128), lambda i: (0, 0)),
        ],
        out_specs=pl.BlockSpec((blk, 128), lambda i: (i, 0)),
        out_shape=jax.ShapeDtypeStruct((n, 128), _F32),
    )(partials, b)


def _wid():
    return lax.axis_index("s") * NC + lax.axis_index("c")


def _lane_sum(v):
    for stride in (8, 4, 2, 1):
        idx = lax.iota(jnp.int32, 16) ^ stride
        v = v + v.at[idx].get(mode="promise_in_bounds")
    return v


def _zero_shared(z_hbm, shared, sid):
    pltpu.sync_copy(z_hbm, shared.at[pl.ds(sid * RA, RA)])

    @pl.when(sid == NS - 1)
    def _tail():
        pltpu.sync_copy(z_hbm.at[pl.ds(0, TAIL)],
                        shared.at[pl.ds(NS * RA, TAIL)])


def _dump_shared(shared, out_hbm, cid, sid):
    r0 = sid * RA
    pltpu.sync_copy(shared.at[pl.ds(r0, RA)], out_hbm.at[cid, pl.ds(r0, RA)])

    @pl.when(sid == NS - 1)
    def _tail():
        pltpu.sync_copy(shared.at[pl.ds(NS * RA, TAIL)],
                        out_hbm.at[cid, pl.ds(NS * RA, TAIL)])


K1 = 40
K2A = 40
K2B = 16
KD = 80


@functools.partial(
    pl.kernel,
    out_type=[
        jax.ShapeDtypeStruct((NC, N, 128), _F32),
        jax.ShapeDtypeStruct((E * 16,), _F32),
    ],
    mesh=_MESH,
    scratch_types=[
        pltpu.VMEM((K1,), jnp.int32),
        pltpu.VMEM((K1,), jnp.int32),
        pltpu.VMEM((K1, 128), _F32),
        pltpu.VMEM((K1, 128), _F32),
        pltpu.VMEM((K1, 128), _F32),
        pltpu.VMEM((K1 * 16,), _F32),
        pltpu.VMEM((H, 16), _F32),
        pltpu.VMEM_SHARED((N, 128), _F32),
        pltpu.SemaphoreType.DMA,
        pltpu.SemaphoreType.DMA,
    ],
)
def _edge16(xl_hbm, xr_hbm, src_hbm, dst_hbm, att_hbm, z128_hbm,
            numer_out, ex_out,
            sidx, didx, xlb, xrb, msgb, exb, attv, snum, sem1, sem2):
    cid = lax.axis_index("c")
    sid = lax.axis_index("s")
    wid = _wid()

    _zero_shared(z128_hbm, snum, sid)
    pltpu.sync_copy(att_hbm, attv)
    plsc.subcore_barrier()

    lane = lax.iota(jnp.int32, 16)

    def chunk(j, carry):
        base = wid * EPW + j * K1
        pltpu.sync_copy(src_hbm.at[pl.ds(base, K1)], sidx)
        pltpu.sync_copy(dst_hbm.at[pl.ds(base, K1)], didx)
        cp1 = pltpu.async_copy(xl_hbm.at[sidx], xlb, sem1)
        cp2 = pltpu.async_copy(xr_hbm.at[didx], xrb, sem2)
        cp1.wait()
        cp2.wait()

        @plsc.parallel_loop(0, K1, unroll=2)
        def edge(e):
            dv = jnp.zeros((16,), _F32)
            for h in range(H):
                a = xlb[e, pl.ds(h * 16, 16)]
                r = xrb[e, pl.ds(h * 16, 16)]
                t = a + r
                t = jnp.maximum(t, 0.2 * t)
                ev = jnp.exp(_lane_sum(attv[h, :] * t))
                msgb[e, pl.ds(h * 16, 16)] = ev * a
                dv = dv + jnp.where(lane == h, ev, 0.0)
            exb[pl.ds(e * 16, 16)] = dv
        pltpu.sync_copy(msgb, snum.at[didx], add=True)
        pltpu.sync_copy(exb, ex_out.at[pl.ds(base * 16, K1 * 16)])
        return carry

    lax.fori_loop(0, EPW // K1, chunk, 0)
    plsc.subcore_barrier()

    _dump_shared(snum, numer_out, cid, sid)


@functools.partial(
    pl.kernel,
    out_type=jax.ShapeDtypeStruct((NC, N, 128), _F32),
    mesh=_MESH,
    scratch_types=[
        pltpu.VMEM((KD,), jnp.int32),
        pltpu.VMEM((KD * 16,), _F32),
        pltpu.VMEM((KD, 128), _F32),
        pltpu.VMEM_SHARED((N, 128), _F32),
        pltpu.SemaphoreType.DMA,
    ],
)
def _denacc(dst_hbm, ex_hbm, z128_hbm,
            denom_out,
            didx, exb, dwb, sden, sem1):
    cid = lax.axis_index("c")
    sid = lax.axis_index("s")
    wid = _wid()

    _zero_shared(z128_hbm, sden, sid)

    zero16 = jnp.zeros((16,), _F32)

    @plsc.parallel_loop(0, KD, unroll=4)
    def zrow(e):
        for cb in range(8):
            dwb[e, pl.ds(cb * 16, 16)] = zero16
    plsc.subcore_barrier()

    def chunk(j, carry):
        base = wid * EPW + j * KD
        pltpu.sync_copy(dst_hbm.at[pl.ds(base, KD)], didx)
        pltpu.sync_copy(ex_hbm.at[pl.ds(base * 16, KD * 16)], exb)

        @plsc.parallel_loop(0, KD, unroll=4)
        def edge(e):
            dwb[e, pl.ds(0, 16)] = exb[pl.ds(e * 16, 16)]
        pltpu.sync_copy(dwb, sden.at[didx], add=True)
        return carry

    lax.fori_loop(0, EPW // KD, chunk, 0)
    plsc.subcore_barrier()

    _dump_shared(sden, denom_out, cid, sid)


@functools.partial(
    pl.kernel,
    out_type=jax.ShapeDtypeStruct((E * 16,), _F32),
    mesh=_MESH,
    scratch_types=[
        pltpu.VMEM((K2A,), jnp.int32),
        pltpu.VMEM((K2A,), jnp.int32),
        pltpu.VMEM((K2A, 1024), _F32),
        pltpu.VMEM((K2A, 1024), _F32),
        pltpu.VMEM((K2A * 16,), _F32),
        pltpu.VMEM((H, 128), _F32),
        pltpu.SemaphoreType.DMA,
        pltpu.SemaphoreType.DMA,
    ],
)
def _edge2a(xl_hbm, xr_hbm, src_hbm, dst_hbm, att_hbm,
            ex_out,
            sidx, didx, xlb, xrb, exb, attv, sem1, sem2):
    wid = _wid()

    pltpu.sync_copy(att_hbm, attv)

    lane = lax.iota(jnp.int32, 16)

    def chunk(j, carry):
        base = wid * EPW + j * K2A
        pltpu.sync_copy(src_hbm.at[pl.ds(base, K2A)], sidx)
        pltpu.sync_copy(dst_hbm.at[pl.ds(base, K2A)], didx)
        cp1 = pltpu.async_copy(xl_hbm.at[sidx], xlb, sem1)
        cp2 = pltpu.async_copy(xr_hbm.at[didx], xrb, sem2)
        cp1.wait()
        cp2.wait()

        @plsc.parallel_loop(0, K2A, unroll=2)
        def edge(e):
            dv = jnp.zeros((16,), _F32)
            for h in range(H):
                acc = jnp.zeros((16,), _F32)
                for cb in range(8):
                    o = h * 128 + cb * 16
                    a = xlb[e, pl.ds(o, 16)]
                    r = xrb[e, pl.ds(o, 16)]
                    t = a + r
                    t = jnp.maximum(t, 0.2 * t)
                    acc = acc + attv[h, pl.ds(cb * 16, 16)] * t
                ev = jnp.exp(_lane_sum(acc))
                dv = dv + jnp.where(lane == h, ev, 0.0)
            exb[pl.ds(e * 16, 16)] = dv
        pltpu.sync_copy(exb, ex_out.at[pl.ds(base * 16, K2A * 16)])
        return carry

    lax.fori_loop(0, EPW // K2A, chunk, 0)


@functools.partial(
    pl.kernel,
    out_type=jax.ShapeDtypeStruct((NC, N, 128), _F32),
    mesh=_MESH,
    scratch_types=[
        pltpu.VMEM((K2B,), jnp.int32),
        pltpu.VMEM((K2B,), jnp.int32),
        pltpu.VMEM((K2B, 1024), _F32),
        pltpu.VMEM((K2B * 16,), _F32),
        pltpu.VMEM((K2B, 128), _F32),
        pltpu.VMEM((K2B, 128), _F32),
        pltpu.VMEM_SHARED((N, 128), _F32),
        pltpu.SemaphoreType.DMA,
        pltpu.SemaphoreType.DMA,
    ],
)
def _edge2b(xl_hbm, src_hbm, dst_hbm, ex_hbm, dn_hbm, z128_hbm,
            out_hbm,
            sidx, didx, xlb, exb, dnb, msgb, sout, sem1, sem2):
    cid = lax.axis_index("c")
    sid = lax.axis_index("s")
    wid = _wid()

    _zero_shared(z128_hbm, sout, sid)
    plsc.subcore_barrier()

    def chunk(j, carry):
        base = wid * EPW + j * K2B
        pltpu.sync_copy(src_hbm.at[pl.ds(base, K2B)], sidx)
        pltpu.sync_copy(dst_hbm.at[pl.ds(base, K2B)], didx)
        cp1 = pltpu.async_copy(xl_hbm.at[sidx], xlb, sem1)
        cp2 = pltpu.async_copy(dn_hbm.at[didx], dnb, sem2)
        pltpu.sync_copy(ex_hbm.at[pl.ds(base * 16, K2B * 16)], exb)
        cp1.wait()
        cp2.wait()

        @plsc.parallel_loop(0, K2B, unroll=2)
        def edge(e):
            ex = exb[pl.ds(e * 16, 16)]
            dn = dnb[e, pl.ds(0, 16)] + _EPS
            w = ex / dn
            m = [jnp.zeros((16,), _F32) for _ in range(8)]
            for h in range(H):
                whv = jnp.broadcast_to(w[h], (16,))
                for cb in range(8):
                    m[cb] = m[cb] + whv * xlb[e, pl.ds(h * 128 + cb * 16, 16)]
            for cb in range(8):
                msgb[e, pl.ds(cb * 16, 16)] = m[cb]
        pltpu.sync_copy(msgb, sout.at[didx], add=True)
        return carry

    lax.fori_loop(0, EPW // K2B, chunk, 0)
    plsc.subcore_barrier()

    _dump_shared(sout, out_hbm, cid, sid)


def kernel(x, edge_index, Wl0, Wr0, att0, b0, Wl1, Wr1, att1, b1,
           Wl2, Wr2, att2, b2):
    src = edge_index[0]
    dst = edge_index[1]

    r = (jnp.arange(128)[None, :] // 16 ==
         jnp.arange(16)[:, None]).astype(_F32)
    z128 = jnp.zeros((RA, 128), _F32)
    b0r = b0.reshape(1, 128)
    b1r = b1.reshape(1, 128)
    b2r = b2.reshape(1, 128)

    xl0, xr0 = _project(x, Wl0, Wr0)
    n0, ex0 = _edge16(xl0, xr0, src, dst, att0, z128)
    d0 = _denacc(dst, ex0, z128)
    xl1, xr1, hin1 = _combine_project(n0, d0, r, b0r, Wl1, Wr1)
    n1, ex1 = _edge16(xl1, xr1, src, dst, att1, z128)
    d1 = _denacc(dst, ex1, z128)
    xl2, xr2 = _combine_res_project(n1, d1, r, b1r, hin1, Wl2, Wr2)
    ex2 = _edge2a(xl2, xr2, src, dst, att2)
    d2 = _denacc(dst, ex2, z128)
    dn2 = _dsum(d2)
    outp = _edge2b(xl2, src, dst, ex2, dn2, z128)
    return _final(outp, b2r)


# speedup vs baseline: 24.4241x; 1.3032x over previous
import functools

import jax
import jax.numpy as jnp
from jax import lax
from jax.experimental import pallas as pl
from jax.experimental.pallas import tpu as pltpu
from jax.experimental.pallas import tpu_sc as plsc

N = 10000
E = 320000
D = 128
H = 8

NC = 2
NS = 16
NW = NC * NS
EPW = E // NW
RA = 624
TAIL = N - NS * RA

_MESH = plsc.VectorSubcoreMesh(
    core_axis_name="c", subcore_axis_name="s", num_cores=NC, num_subcores=NS)

_EPS = 1e-16
_F32 = jnp.float32


def _proj_body(x_ref, wl_ref, wr_ref, xl_ref, xr_ref):
    xv = x_ref[...]
    xl_ref[...] = jnp.dot(xv, wl_ref[...], preferred_element_type=_F32)
    xr_ref[...] = jnp.dot(xv, wr_ref[...], preferred_element_type=_F32)


def _project(x, Wl, Wr, blk=1000):
    n, din = x.shape
    dout = Wl.shape[1]
    return pl.pallas_call(
        _proj_body,
        grid=(n // blk,),
        in_specs=[
            pl.BlockSpec((blk, din), lambda i: (i, 0)),
            pl.BlockSpec((din, dout), lambda i: (0, 0)),
            pl.BlockSpec((din, dout), lambda i: (0, 0)),
        ],
        out_specs=[
            pl.BlockSpec((blk, dout), lambda i: (i, 0)),
            pl.BlockSpec((blk, dout), lambda i: (i, 0)),
        ],
        out_shape=[
            jax.ShapeDtypeStruct((n, dout), _F32),
            jax.ShapeDtypeStruct((n, dout), _F32),
        ],
    )(x, Wl, Wr)


def _act(numer2, denom2, r, b):
    nsum = numer2[0] + numer2[1]
    dsum = jnp.dot((denom2[0] + denom2[1])[:, :16], r,
                   preferred_element_type=_F32) + _EPS
    hv = nsum / dsum + b
    return jnp.where(hv > 0, hv, jnp.exp(hv) - 1.0)


def _comb_proj_body(n_ref, d_ref, r_ref, b_ref, wl_ref, wr_ref,
                    xl_ref, xr_ref, h_ref):
    hv = _act(n_ref[...], d_ref[...], r_ref[...], b_ref[...])
    h_ref[...] = hv
    xl_ref[...] = jnp.dot(hv, wl_ref[...], preferred_element_type=_F32)
    xr_ref[...] = jnp.dot(hv, wr_ref[...], preferred_element_type=_F32)


def _combine_project(numer, denom, r, b, Wl, Wr, blk=1000):
    n = numer.shape[1]
    din, dout = Wl.shape
    return pl.pallas_call(
        _comb_proj_body,
        grid=(n // blk,),
        in_specs=[
            pl.BlockSpec((2, blk, 128), lambda i: (0, i, 0)),
            pl.BlockSpec((2, blk, 128), lambda i: (0, i, 0)),
            pl.BlockSpec((16, 128), lambda i: (0, 0)),
            pl.BlockSpec((1, 128), lambda i: (0, 0)),
            pl.BlockSpec((din, dout), lambda i: (0, 0)),
            pl.BlockSpec((din, dout), lambda i: (0, 0)),
        ],
        out_specs=[
            pl.BlockSpec((blk, dout), lambda i: (i, 0)),
            pl.BlockSpec((blk, dout), lambda i: (i, 0)),
            pl.BlockSpec((blk, 128), lambda i: (i, 0)),
        ],
        out_shape=[
            jax.ShapeDtypeStruct((n, dout), _F32),
            jax.ShapeDtypeStruct((n, dout), _F32),
            jax.ShapeDtypeStruct((n, 128), _F32),
        ],
    )(numer, denom, r, b, Wl, Wr)


def _comb_res_proj_body(n_ref, d_ref, r_ref, b_ref, res_ref, wl_ref, wr_ref,
                        xl_ref, xr_ref):
    hv = _act(n_ref[...], d_ref[...], r_ref[...], b_ref[...]) + res_ref[...]
    xl_ref[...] = jnp.dot(hv, wl_ref[...], preferred_element_type=_F32)
    xr_ref[...] = jnp.dot(hv, wr_ref[...], preferred_element_type=_F32)


def _combine_res_project(numer, denom, r, b, res, Wl, Wr, blk=1000):
    n = numer.shape[1]
    din, dout = Wl.shape
    return pl.pallas_call(
        _comb_res_proj_body,
        grid=(n // blk,),
        in_specs=[
            pl.BlockSpec((2, blk, 128), lambda i: (0, i, 0)),
            pl.BlockSpec((2, blk, 128), lambda i: (0, i, 0)),
            pl.BlockSpec((16, 128), lambda i: (0, 0)),
            pl.BlockSpec((1, 128), lambda i: (0, 0)),
            pl.BlockSpec((blk, 128), lambda i: (i, 0)),
            pl.BlockSpec((din, dout), lambda i: (0, 0)),
            pl.BlockSpec((din, dout), lambda i: (0, 0)),
        ],
        out_specs=[
            pl.BlockSpec((blk, dout), lambda i: (i, 0)),
            pl.BlockSpec((blk, dout), lambda i: (i, 0)),
        ],
        out_shape=[
            jax.ShapeDtypeStruct((n, dout), _F32),
            jax.ShapeDtypeStruct((n, dout), _F32),
        ],
    )(numer, denom, r, b, res, Wl, Wr)


def _dsum_body(d_ref, o_ref):
    o_ref[...] = d_ref[0] + d_ref[1]


def _dsum(denom, blk=1000):
    n = denom.shape[1]
    return pl.pallas_call(
        _dsum_body,
        grid=(n // blk,),
        in_specs=[pl.BlockSpec((2, blk, 128), lambda i: (0, i, 0))],
        out_specs=pl.BlockSpec((blk, 128), lambda i: (i, 0)),
        out_shape=jax.ShapeDtypeStruct((n, 128), _F32),
    )(denom)


def _final_body(p_ref, b_ref, o_ref):
    o_ref[...] = (p_ref[0] + p_ref[1]) * (1.0 / H) + b_ref[...]


def _final(partials, b, blk=1000):
    n = partials.shape[1]
    return pl.pallas_call(
        _final_body,
        grid=(n // blk,),
        in_specs=[
            pl.BlockSpec((2, blk, 128), lambda i: (0, i, 0)),
            pl.BlockSpec((1, 128), lambda i: (0, 0)),
        ],
        out_specs=pl.BlockSpec((blk, 128), lambda i: (i, 0)),
        out_shape=jax.ShapeDtypeStruct((n, 128), _F32),
    )(partials, b)


def _wid():
    return lax.axis_index("s") * NC + lax.axis_index("c")


def _lane_sum(v):
    for stride in (8, 4, 2, 1):
        idx = lax.iota(jnp.int32, 16) ^ stride
        v = v + v.at[idx].get(mode="promise_in_bounds")
    return v


def _zero_shared(z_hbm, shared, sid):
    pltpu.sync_copy(z_hbm, shared.at[pl.ds(sid * RA, RA)])

    @pl.when(sid == NS - 1)
    def _tail():
        pltpu.sync_copy(z_hbm.at[pl.ds(0, TAIL)],
                        shared.at[pl.ds(NS * RA, TAIL)])


def _dump_shared(shared, out_hbm, cid, sid):
    r0 = sid * RA
    pltpu.sync_copy(shared.at[pl.ds(r0, RA)], out_hbm.at[cid, pl.ds(r0, RA)])

    @pl.when(sid == NS - 1)
    def _tail():
        pltpu.sync_copy(shared.at[pl.ds(NS * RA, TAIL)],
                        out_hbm.at[cid, pl.ds(NS * RA, TAIL)])


K1 = 40
K2A = 16
K2B = 16
KD = 80


@functools.partial(
    pl.kernel,
    out_type=[
        jax.ShapeDtypeStruct((NC, N, 128), _F32),
        jax.ShapeDtypeStruct((E * 16,), _F32),
    ],
    mesh=_MESH,
    scratch_types=(
        [pltpu.VMEM((K1,), jnp.int32)] * 4 +
        [pltpu.VMEM((K1, 128), _F32)] * 4 +
        [pltpu.VMEM((K1, 128), _F32),
         pltpu.VMEM((K1 * 16,), _F32),
         pltpu.VMEM((H, 16), _F32),
         pltpu.VMEM_SHARED((N, 128), _F32)] +
        [pltpu.SemaphoreType.DMA] * 2
    ),
)
def _edge16(xl_hbm, xr_hbm, src_hbm, dst_hbm, att_hbm, z128_hbm,
            numer_out, ex_out,
            sgi0, sgi1, dgi0, dgi1, xlb0, xlb1, xrb0, xrb1,
            msgb, exb, attv, snum, gsem0, gsem1):
    cid = lax.axis_index("c")
    sid = lax.axis_index("s")
    wid = _wid()
    sgi = [sgi0, sgi1]
    dgi = [dgi0, dgi1]
    xlb = [xlb0, xlb1]
    xrb = [xrb0, xrb1]
    gsem = [gsem0, gsem1]
    e0 = wid * EPW
    nch = EPW // K1

    _zero_shared(z128_hbm, snum, sid)
    pltpu.sync_copy(att_hbm, attv)
    plsc.subcore_barrier()

    lane = lax.iota(jnp.int32, 16)

    def load_idx(jj, b):
        base = e0 + jj * K1
        pltpu.sync_copy(src_hbm.at[pl.ds(base, K1)], sgi[b])
        pltpu.sync_copy(dst_hbm.at[pl.ds(base, K1)], dgi[b])

    def gissue(b):
        pltpu.async_copy(xl_hbm.at[sgi[b]], xlb[b], gsem[b])
        pltpu.async_copy(xr_hbm.at[dgi[b]], xrb[b], gsem[b])

    def gwait(b):
        pltpu.make_async_copy(xl_hbm.at[sgi[b]], xlb[b], gsem[b]).wait()
        pltpu.make_async_copy(xr_hbm.at[dgi[b]], xrb[b], gsem[b]).wait()

    load_idx(0, 0)
    gissue(0)

    def pair(j2, carry):
        for b in (0, 1):
            jj = j2 * 2 + b
            bb = 1 - b

            @pl.when(jj + 1 < nch)
            def _p(jj=jj, bb=bb):
                load_idx(jj + 1, bb)
                gissue(bb)

            gwait(b)

            @plsc.parallel_loop(0, K1, unroll=2)
            def edge(e, b=b):
                dv = jnp.zeros((16,), _F32)
                for h in range(H):
                    a = xlb[b][e, pl.ds(h * 16, 16)]
                    r = xrb[b][e, pl.ds(h * 16, 16)]
                    t = a + r
                    t = jnp.maximum(t, 0.2 * t)
                    ev = jnp.exp(_lane_sum(attv[h, :] * t))
                    msgb[e, pl.ds(h * 16, 16)] = ev * a
                    dv = dv + jnp.where(lane == h, ev, 0.0)
                exb[pl.ds(e * 16, 16)] = dv

            base = e0 + jj * K1
            pltpu.sync_copy(msgb, snum.at[dgi[b]], add=True)
            pltpu.sync_copy(exb, ex_out.at[pl.ds(base * 16, K1 * 16)])
        return carry

    lax.fori_loop(0, nch // 2, pair, 0)
    plsc.subcore_barrier()

    _dump_shared(snum, numer_out, cid, sid)


@functools.partial(
    pl.kernel,
    out_type=jax.ShapeDtypeStruct((NC, N, 128), _F32),
    mesh=_MESH,
    scratch_types=[
        pltpu.VMEM((KD,), jnp.int32),
        pltpu.VMEM((KD * 16,), _F32),
        pltpu.VMEM((KD, 128), _F32),
        pltpu.VMEM_SHARED((N, 128), _F32),
        pltpu.SemaphoreType.DMA,
    ],
)
def _denacc(dst_hbm, ex_hbm, z128_hbm,
            denom_out,
            didx, exb, dwb, sden, sem1):
    cid = lax.axis_index("c")
    sid = lax.axis_index("s")
    wid = _wid()

    _zero_shared(z128_hbm, sden, sid)

    zero16 = jnp.zeros((16,), _F32)

    @plsc.parallel_loop(0, KD, unroll=4)
    def zrow(e):
        for cb in range(8):
            dwb[e, pl.ds(cb * 16, 16)] = zero16
    plsc.subcore_barrier()

    def chunk(j, carry):
        base = wid * EPW + j * KD
        pltpu.sync_copy(dst_hbm.at[pl.ds(base, KD)], didx)
        pltpu.sync_copy(ex_hbm.at[pl.ds(base * 16, KD * 16)], exb)

        @plsc.parallel_loop(0, KD, unroll=4)
        def edge(e):
            dwb[e, pl.ds(0, 16)] = exb[pl.ds(e * 16, 16)]
        pltpu.sync_copy(dwb, sden.at[didx], add=True)
        return carry

    lax.fori_loop(0, EPW // KD, chunk, 0)
    plsc.subcore_barrier()

    _dump_shared(sden, denom_out, cid, sid)


@functools.partial(
    pl.kernel,
    out_type=jax.ShapeDtypeStruct((E * 16,), _F32),
    mesh=_MESH,
    scratch_types=(
        [pltpu.VMEM((K2A,), jnp.int32)] * 4 +
        [pltpu.VMEM((K2A, 1024), _F32)] * 4 +
        [pltpu.VMEM((K2A * 16,), _F32),
         pltpu.VMEM((H, 128), _F32)] +
        [pltpu.SemaphoreType.DMA] * 2
    ),
)
def _edge2a(xl_hbm, xr_hbm, src_hbm, dst_hbm, att_hbm,
            ex_out,
            sgi0, sgi1, dgi0, dgi1, xlb0, xlb1, xrb0, xrb1,
            exb, attv, gsem0, gsem1):
    wid = _wid()
    sgi = [sgi0, sgi1]
    dgi = [dgi0, dgi1]
    xlb = [xlb0, xlb1]
    xrb = [xrb0, xrb1]
    gsem = [gsem0, gsem1]
    e0 = wid * EPW
    nch = EPW // K2A

    pltpu.sync_copy(att_hbm, attv)

    lane = lax.iota(jnp.int32, 16)

    def load_idx(jj, b):
        base = e0 + jj * K2A
        pltpu.sync_copy(src_hbm.at[pl.ds(base, K2A)], sgi[b])
        pltpu.sync_copy(dst_hbm.at[pl.ds(base, K2A)], dgi[b])

    def gissue(b):
        pltpu.async_copy(xl_hbm.at[sgi[b]], xlb[b], gsem[b])
        pltpu.async_copy(xr_hbm.at[dgi[b]], xrb[b], gsem[b])

    def gwait(b):
        pltpu.make_async_copy(xl_hbm.at[sgi[b]], xlb[b], gsem[b]).wait()
        pltpu.make_async_copy(xr_hbm.at[dgi[b]], xrb[b], gsem[b]).wait()

    def step(jj, b):
        @plsc.parallel_loop(0, K2A, unroll=2)
        def edge(e):
            dv = jnp.zeros((16,), _F32)
            for h in range(H):
                acc = jnp.zeros((16,), _F32)
                for cb in range(8):
                    o = h * 128 + cb * 16
                    a = xlb[b][e, pl.ds(o, 16)]
                    r = xrb[b][e, pl.ds(o, 16)]
                    t = a + r
                    t = jnp.maximum(t, 0.2 * t)
                    acc = acc + attv[h, pl.ds(cb * 16, 16)] * t
                ev = jnp.exp(_lane_sum(acc))
                dv = dv + jnp.where(lane == h, ev, 0.0)
            exb[pl.ds(e * 16, 16)] = dv

        base = e0 + jj * K2A
        pltpu.sync_copy(exb, ex_out.at[pl.ds(base * 16, K2A * 16)])

    load_idx(0, 0)
    gissue(0)

    def pair(j2, carry):
        for b in (0, 1):
            jj = j2 * 2 + b
            bb = 1 - b

            @pl.when(jj + 1 < nch)
            def _p(jj=jj, bb=bb):
                load_idx(jj + 1, bb)
                gissue(bb)

            gwait(b)
            step(jj, b)
        return carry

    lax.fori_loop(0, (nch - 1) // 2, pair, 0)
    gwait(0)
    step(nch - 1, 0)


@functools.partial(
    pl.kernel,
    out_type=jax.ShapeDtypeStruct((NC, N, 128), _F32),
    mesh=_MESH,
    scratch_types=(
        [pltpu.VMEM((K2B,), jnp.int32)] * 4 +
        [pltpu.VMEM((K2B, 1024), _F32)] * 2 +
        [pltpu.VMEM((K2B * 16,), _F32)] * 2 +
        [pltpu.VMEM((K2B, 128), _F32)] * 2 +
        [pltpu.VMEM((K2B, 128), _F32),
         pltpu.VMEM_SHARED((N, 128), _F32)] +
        [pltpu.SemaphoreType.DMA] * 2
    ),
)
def _edge2b(xl_hbm, src_hbm, dst_hbm, ex_hbm, dn_hbm, z128_hbm,
            out_hbm,
            sgi0, sgi1, dgi0, dgi1, xlb0, xlb1, exb0, exb1, dnb0, dnb1,
            msgb, sout, gsem0, gsem1):
    cid = lax.axis_index("c")
    sid = lax.axis_index("s")
    wid = _wid()
    sgi = [sgi0, sgi1]
    dgi = [dgi0, dgi1]
    xlb = [xlb0, xlb1]
    exb = [exb0, exb1]
    dnb = [dnb0, dnb1]
    gsem = [gsem0, gsem1]
    e0 = wid * EPW
    nch = EPW // K2B

    _zero_shared(z128_hbm, sout, sid)
    plsc.subcore_barrier()

    def load_idx(jj, b):
        base = e0 + jj * K2B
        pltpu.sync_copy(src_hbm.at[pl.ds(base, K2B)], sgi[b])
        pltpu.sync_copy(dst_hbm.at[pl.ds(base, K2B)], dgi[b])

    def gissue(jj, b):
        base = e0 + jj * K2B
        pltpu.async_copy(xl_hbm.at[sgi[b]], xlb[b], gsem[b])
        pltpu.async_copy(dn_hbm.at[dgi[b]], dnb[b], gsem[b])
        pltpu.async_copy(ex_hbm.at[pl.ds(base * 16, K2B * 16)], exb[b],
                         gsem[b])

    def gwait(jj, b):
        base = e0 + jj * K2B
        pltpu.make_async_copy(xl_hbm.at[sgi[b]], xlb[b], gsem[b]).wait()
        pltpu.make_async_copy(dn_hbm.at[dgi[b]], dnb[b], gsem[b]).wait()
        pltpu.make_async_copy(ex_hbm.at[pl.ds(base * 16, K2B * 16)], exb[b],
                              gsem[b]).wait()

    def step(b):
        @plsc.parallel_loop(0, K2B, unroll=2)
        def edge(e):
            ex = exb[b][pl.ds(e * 16, 16)]
            dn = dnb[b][e, pl.ds(0, 16)] + _EPS
            w = ex / dn
            m = [jnp.zeros((16,), _F32) for _ in range(8)]
            for h in range(H):
                whv = jnp.broadcast_to(w[h], (16,))
                for cb in range(8):
                    m[cb] = (m[cb] +
                             whv * xlb[b][e, pl.ds(h * 128 + cb * 16, 16)])
            for cb in range(8):
                msgb[e, pl.ds(cb * 16, 16)] = m[cb]

        pltpu.sync_copy(msgb, sout.at[dgi[b]], add=True)

    load_idx(0, 0)
    gissue(0, 0)

    def pair(j2, carry):
        for b in (0, 1):
            jj = j2 * 2 + b
            bb = 1 - b

            @pl.when(jj + 1 < nch)
            def _p(jj=jj, bb=bb):
                load_idx(jj + 1, bb)
                gissue(jj + 1, bb)

            gwait(jj, b)
            step(b)
        return carry

    lax.fori_loop(0, (nch - 1) // 2, pair, 0)
    gwait(nch - 1, 0)
    step(0)
    plsc.subcore_barrier()

    _dump_shared(sout, out_hbm, cid, sid)


def kernel(x, edge_index, Wl0, Wr0, att0, b0, Wl1, Wr1, att1, b1,
           Wl2, Wr2, att2, b2):
    src = edge_index[0]
    dst = edge_index[1]

    r = (jnp.arange(128)[None, :] // 16 ==
         jnp.arange(16)[:, None]).astype(_F32)
    z128 = jnp.zeros((RA, 128), _F32)
    b0r = b0.reshape(1, 128)
    b1r = b1.reshape(1, 128)
    b2r = b2.reshape(1, 128)

    xl0, xr0 = _project(x, Wl0, Wr0)
    n0, ex0 = _edge16(xl0, xr0, src, dst, att0, z128)
    d0 = _denacc(dst, ex0, z128)
    xl1, xr1, hin1 = _combine_project(n0, d0, r, b0r, Wl1, Wr1)
    n1, ex1 = _edge16(xl1, xr1, src, dst, att1, z128)
    d1 = _denacc(dst, ex1, z128)
    xl2, xr2 = _combine_res_project(n1, d1, r, b1r, hin1, Wl2, Wr2)
    ex2 = _edge2a(xl2, xr2, src, dst, att2)
    d2 = _denacc(dst, ex2, z128)
    dn2 = _dsum(d2)
    outp = _edge2b(xl2, src, dst, ex2, dn2, z128)
    return _final(outp, b2r)
